# baseline (speedup 1.0000x reference)
import functools

import jax
import jax.numpy as jnp
from jax import lax
from jax.experimental import pallas as pl
from jax.experimental.pallas import tpu as pltpu
from jax.experimental.pallas import tpu_sc as plsc

_LANES = 16

_mesh = plsc.VectorSubcoreMesh(core_axis_name="c", subcore_axis_name="s")


@functools.partial(
    pl.kernel,
    mesh=_mesh,
    out_type=jax.ShapeDtypeStruct((_LANES,), jnp.float32),
    scratch_types=[
        pltpu.VMEM((_LANES,), jnp.float32),
        pltpu.VMEM((_LANES,), jnp.float32),
    ],
)
def _add16(x_hbm, y_hbm, o_hbm, xv, yv):
    @pl.when((lax.axis_index("c") == 0) & (lax.axis_index("s") == 0))
    def _():
        pltpu.sync_copy(x_hbm, xv)
        pltpu.sync_copy(y_hbm, yv)
        xv[...] = xv[...] + yv[...]
        pltpu.sync_copy(xv, o_hbm)


def kernel(Y, X):
    x = jnp.pad(X.reshape(4), (0, _LANES - 4))
    y = jnp.pad(Y.reshape(4), (0, _LANES - 4))
    out = _add16(x, y)
    return out[:4].reshape(2, 2)


# speedup vs baseline: 1.0865x; 1.0865x over previous
import functools

import jax
import jax.numpy as jnp
from jax import lax
from jax.experimental import pallas as pl
from jax.experimental.pallas import tpu as pltpu
from jax.experimental.pallas import tpu_sc as plsc

_LANES = 16

_mesh = plsc.VectorSubcoreMesh(
    core_axis_name="c", subcore_axis_name="s", num_cores=1
)


@functools.partial(
    pl.kernel,
    mesh=_mesh,
    out_type=jax.ShapeDtypeStruct((4,), jnp.float32),
    scratch_types=[
        pltpu.VMEM((_LANES,), jnp.float32),
        pltpu.VMEM((_LANES,), jnp.float32),
    ],
)
def _add4(x_hbm, y_hbm, o_hbm, xv, yv):
    @pl.when((lax.axis_index("c") == 0) & (lax.axis_index("s") == 0))
    def _():
        pltpu.sync_copy(x_hbm, xv.at[pl.ds(0, 4)])
        pltpu.sync_copy(y_hbm, yv.at[pl.ds(0, 4)])
        xv[...] = xv[...] + yv[...]
        pltpu.sync_copy(xv.at[pl.ds(0, 4)], o_hbm)


def kernel(Y, X):
    return _add4(X.reshape(4), Y.reshape(4)).reshape(2, 2)


# speedup vs baseline: 1.1620x; 1.0695x over previous
import functools

import jax
import jax.numpy as jnp
from jax import lax
from jax.experimental import pallas as pl
from jax.experimental.pallas import tpu as pltpu
from jax.experimental.pallas import tpu_sc as plsc

_mesh = plsc.ScalarSubcoreMesh(axis_name="c", num_cores=1)


@functools.partial(
    pl.kernel,
    mesh=_mesh,
    out_type=jax.ShapeDtypeStruct((4,), jnp.float32),
    scratch_types=[
        pltpu.SMEM((4,), jnp.float32),
        pltpu.SMEM((4,), jnp.float32),
    ],
)
def _add4(x_hbm, y_hbm, o_hbm, xs, ys):
    @pl.when(lax.axis_index("c") == 0)
    def _():
        pltpu.sync_copy(x_hbm, xs)
        pltpu.sync_copy(y_hbm, ys)
        for i in range(4):
            xs[i] = xs[i] + ys[i]
        pltpu.sync_copy(xs, o_hbm)


def kernel(Y, X):
    return _add4(X.reshape(4), Y.reshape(4)).reshape(2, 2)


# speedup vs baseline: 1.2239x; 1.0533x over previous
import functools

import jax
import jax.numpy as jnp
from jax import lax
from jax.experimental import pallas as pl
from jax.experimental.pallas import tpu as pltpu
from jax.experimental.pallas import tpu_sc as plsc

_mesh = plsc.ScalarSubcoreMesh(axis_name="c", num_cores=1)


@functools.partial(
    pl.kernel,
    mesh=_mesh,
    out_type=jax.ShapeDtypeStruct((2, 2), jnp.float32),
    scratch_types=[
        pltpu.SMEM((2, 2), jnp.float32),
        pltpu.SMEM((2, 2), jnp.float32),
    ],
)
def _add22(x_hbm, y_hbm, o_hbm, xs, ys):
    pltpu.sync_copy(x_hbm, xs)
    pltpu.sync_copy(y_hbm, ys)
    for i in range(2):
        for j in range(2):
            xs[i, j] = xs[i, j] + ys[i, j]
    pltpu.sync_copy(xs, o_hbm)


def kernel(Y, X):
    return _add22(X, Y)


# speedup vs baseline: 1.2891x; 1.0532x over previous
import functools

import jax
import jax.numpy as jnp
from jax import lax
from jax.experimental import pallas as pl
from jax.experimental.pallas import tpu as pltpu
from jax.experimental.pallas import tpu_sc as plsc

_mesh = plsc.ScalarSubcoreMesh(axis_name="c", num_cores=1)


@functools.partial(
    pl.kernel,
    mesh=_mesh,
    out_type=jax.ShapeDtypeStruct((2, 2), jnp.float32),
    scratch_types=[
        pltpu.SMEM((2, 2), jnp.float32),
        pltpu.SMEM((2, 2), jnp.float32),
        pltpu.SemaphoreType.DMA,
        pltpu.SemaphoreType.DMA,
    ],
)
def _add22(x_hbm, y_hbm, o_hbm, xs, ys, sem_x, sem_y):
    cx = pltpu.async_copy(x_hbm, xs, sem_x)
    cy = pltpu.async_copy(y_hbm, ys, sem_y)
    cx.wait()
    cy.wait()
    for i in range(2):
        for j in range(2):
            xs[i, j] = xs[i, j] + ys[i, j]
    pltpu.sync_copy(xs, o_hbm)


def kernel(Y, X):
    return _add22(X, Y)
